# baseline (speedup 1.0000x reference)
import functools

import jax
import jax.numpy as jnp
from jax import lax
from jax.experimental import pallas as pl
from jax.experimental.pallas import tpu as pltpu
from jax.experimental.pallas import tpu_sc as plsc

AVG = 15
MAXW = 3
GRP = AVG * MAXW
LANES = 16
CHUNK = 120
HMASK = -65536


def _encoder_body(nq, n_chunks, out_per_w, subs_per_w, scale_w, scale_p,
                  src_ref, w_ref, p_ref, out_ref,
                  idx2, rows, pbuf, stage, pp, out_v, shared,
                  sem_a, sem_b):
    ns = lax.axis_index("s")
    toks_per_w = subs_per_w * AVG
    tok0 = ns * toks_per_w
    half = n_chunks // 2

    copies = []
    for k in range(n_chunks):
        pltpu.sync_copy(src_ref.at[pl.ds(tok0 + k * CHUNK, CHUNK)],
                        idx2.at[k])
        sem = sem_a if k < half else sem_b
        copies.append(pltpu.async_copy(
            w_ref.at[idx2.at[k]], rows.at[pl.ds(k * CHUNK, CHUNK)], sem))

    g_per_t = subs_per_w // 16
    rows_per_t = g_per_t * AVG
    p0 = ns * rows_per_t
    pltpu.sync_copy(p_ref.at[pl.ds(p0, rows_per_t)], pbuf)
    for t in range(g_per_t):
        base = t * AVG
        for q in range(nq):
            sl = pl.ds(q * LANES, LANES)
            acc = pbuf[base, sl]
            for i in range(1, AVG):
                acc = acc + pbuf[base + i, sl]
            stage[t, sl] = acc * scale_p
    pltpu.sync_copy(stage, shared.at[pl.ds(ns * g_per_t, g_per_t)])
    plsc.subcore_barrier()
    pltpu.sync_copy(shared, pp)

    def compute(j, _):
        r0 = j * GRP
        res = None
        for kk in range(MAXW):
            b0 = r0 + kk * AVG
            accs = [None] * nq
            for i in range(AVG):
                for h in range(nq // 2):
                    w = rows[b0 + i, pl.ds(h * LANES, LANES)]
                    a = plsc.bitcast(w << 16, jnp.float32)
                    b = plsc.bitcast(w & HMASK, jnp.float32)
                    if accs[2 * h] is None:
                        accs[2 * h], accs[2 * h + 1] = a, b
                    else:
                        accs[2 * h] = accs[2 * h] + a
                        accs[2 * h + 1] = accs[2 * h + 1] + b
            es = [accs[q] * scale_w + pp[j * MAXW + kk, pl.ds(q * LANES, LANES)]
                  for q in range(nq)]
            if res is None:
                res = es
            else:
                res = [jnp.maximum(x, y) for x, y in zip(res, es)]
        for q in range(nq):
            out_v[j, pl.ds(q * LANES, LANES)] = res[q]
        return 0

    for k in range(half):
        copies[k].wait()
    j_mid = (half * CHUNK) // GRP
    lax.fori_loop(0, j_mid, compute, 0)

    for k in range(half, n_chunks):
        copies[k].wait()
    lax.fori_loop(j_mid, out_per_w, compute, 0)

    pltpu.sync_copy(out_v, out_ref.at[pl.ds(ns * out_per_w, out_per_w)])


@functools.partial(jax.jit, static_argnums=(3, 4, 5))
def _encode(src_flat, w_packed, p, n_out, d, n_workers):
    out_per_w = n_out // n_workers
    subs_per_w = out_per_w * MAXW
    n_chunks = (subs_per_w * AVG) // CHUNK
    nq = d // LANES
    scale_w = float(d) ** 0.5 / AVG
    scale_p = 1.0 / AVG
    mesh = plsc.VectorSubcoreMesh(core_axis_name="c", subcore_axis_name="s",
                                  num_cores=1)
    body = functools.partial(_encoder_body, nq, n_chunks, out_per_w,
                             subs_per_w, scale_w, scale_p)
    return pl.kernel(
        body,
        out_type=jax.ShapeDtypeStruct((n_out, d), jnp.float32),
        mesh=mesh,
        compiler_params=pltpu.CompilerParams(use_tc_tiling_on_sc=False,
                                             needs_layout_passes=False),
        scratch_types=[
            pltpu.VMEM((n_chunks, CHUNK), jnp.int32),
            pltpu.VMEM((n_chunks * CHUNK, d // 2), jnp.int32),
            pltpu.VMEM((subs_per_w // 16 * AVG, d), jnp.float32),
            pltpu.VMEM((subs_per_w // 16, d), jnp.float32),
            pltpu.VMEM((subs_per_w, d), jnp.float32),
            pltpu.VMEM((out_per_w, d), jnp.float32),
            pltpu.VMEM_SHARED((subs_per_w, d), jnp.float32),
            pltpu.SemaphoreType.DMA,
            pltpu.SemaphoreType.DMA,
        ],
    )(src_flat, w_packed, p)


def kernel(src, W, P):
    b, l = src.shape
    v, d = W.shape
    n_out = b * (l // GRP)
    w_packed = lax.bitcast_convert_type(
        W.astype(jnp.bfloat16).reshape(v, d // 2, 2), jnp.int32)
    p_perm = (P.reshape(-1, d // 32, LANES, 2)
               .transpose(0, 1, 3, 2)
               .reshape(P.shape))
    out = _encode(src.reshape(b * l), w_packed, p_perm, n_out, d, 16)
    out = (out.reshape(n_out, d // 32, 2, LANES)
              .transpose(0, 1, 3, 2)
              .reshape(n_out, d))
    return out.reshape(b, l // GRP, d)


# speedup vs baseline: 1.9933x; 1.9933x over previous
import functools

import jax
import jax.numpy as jnp
from jax import lax
from jax.experimental import pallas as pl
from jax.experimental.pallas import tpu as pltpu
from jax.experimental.pallas import tpu_sc as plsc

AVG = 15
MAXW = 3
GRP = AVG * MAXW
LANES = 16
CHUNK = 120
HIMASK = 0xFFFF0000


def _pack_body(w_ref, out_ref):
    bits = pltpu.bitcast(w_ref[...], jnp.uint32)
    rnd = (bits + 0x7FFF + ((bits >> 16) & 1)) >> 16
    d2 = out_ref.shape[1]
    out_ref[...] = rnd[:, :d2] | (rnd[:, d2:] << 16)


def _pack_table(w, blk):
    v, d = w.shape
    return pl.pallas_call(
        _pack_body,
        grid=(v // blk,),
        in_specs=[pl.BlockSpec((blk, d), lambda i: (i, 0))],
        out_specs=pl.BlockSpec((blk, d // 2), lambda i: (i, 0)),
        out_shape=jax.ShapeDtypeStruct((v, d // 2), jnp.uint32),
    )(w)


def _pool_body(p_ref, out_ref):
    x = p_ref[...]
    n, d = out_ref.shape
    out_ref[...] = x.reshape(n, AVG, d).sum(axis=1) * (1.0 / AVG)


def _pool_pos(p, n_sub, d):
    return pl.pallas_call(
        _pool_body,
        in_specs=[pl.BlockSpec((n_sub * AVG, d), lambda: (0, 0))],
        out_specs=pl.BlockSpec((n_sub, d), lambda: (0, 0)),
        out_shape=jax.ShapeDtypeStruct((n_sub, d), jnp.float32),
    )(p[:n_sub * AVG])


def _encoder_body(nq, n_chunks, out_per_w, scale_w,
                  src_ref, w_ref, pp_ref, out_ref,
                  idx2, rows, pp, out_v, sem_a, sem_b):
    ns = lax.axis_index("s")
    toks_per_w = out_per_w * GRP
    tok0 = ns * toks_per_w
    half = n_chunks // 2

    copies = []
    for k in range(n_chunks):
        pltpu.sync_copy(src_ref.at[pl.ds(tok0 + k * CHUNK, CHUNK)],
                        idx2.at[k])
        sem = sem_a if k < half else sem_b
        copies.append(pltpu.async_copy(
            w_ref.at[idx2.at[k]], rows.at[pl.ds(k * CHUNK, CHUNK)], sem))

    pltpu.sync_copy(pp_ref, pp)

    def compute(j, _):
        r0 = j * GRP
        res = None
        for kk in range(MAXW):
            b0 = r0 + kk * AVG
            accs = [None] * nq
            for i in range(AVG):
                for h in range(nq // 2):
                    w = rows[b0 + i, pl.ds(h * LANES, LANES)]
                    lo = plsc.bitcast(w << 16, jnp.float32)
                    hi = plsc.bitcast(w & jnp.uint32(HIMASK), jnp.float32)
                    if accs[h] is None:
                        accs[h], accs[nq // 2 + h] = lo, hi
                    else:
                        accs[h] = accs[h] + lo
                        accs[nq // 2 + h] = accs[nq // 2 + h] + hi
            es = [accs[q] * scale_w + pp[j * MAXW + kk, pl.ds(q * LANES, LANES)]
                  for q in range(nq)]
            if res is None:
                res = es
            else:
                res = [jnp.maximum(x, y) for x, y in zip(res, es)]
        for q in range(nq):
            out_v[j, pl.ds(q * LANES, LANES)] = res[q]
        return 0

    for k in range(half):
        copies[k].wait()
    j_mid = (half * CHUNK) // GRP
    lax.fori_loop(0, j_mid, compute, 0)

    for k in range(half, n_chunks):
        copies[k].wait()
    lax.fori_loop(j_mid, out_per_w, compute, 0)

    pltpu.sync_copy(out_v, out_ref.at[pl.ds(ns * out_per_w, out_per_w)])


@functools.partial(jax.jit, static_argnums=(3, 4, 5))
def _encode(src_flat, w_packed, pooled_p, n_out, d, n_workers):
    out_per_w = n_out // n_workers
    n_chunks = (out_per_w * GRP) // CHUNK
    nq = d // LANES
    scale_w = float(d) ** 0.5 / AVG
    mesh = plsc.VectorSubcoreMesh(core_axis_name="c", subcore_axis_name="s",
                                  num_cores=1)
    body = functools.partial(_encoder_body, nq, n_chunks, out_per_w, scale_w)
    return pl.kernel(
        body,
        out_type=jax.ShapeDtypeStruct((n_out, d), jnp.float32),
        mesh=mesh,
        compiler_params=pltpu.CompilerParams(use_tc_tiling_on_sc=False,
                                             needs_layout_passes=False),
        scratch_types=[
            pltpu.VMEM((n_chunks, CHUNK), jnp.int32),
            pltpu.VMEM((n_chunks * CHUNK, d // 2), jnp.uint32),
            pltpu.VMEM((out_per_w * MAXW, d), jnp.float32),
            pltpu.VMEM((out_per_w, d), jnp.float32),
            pltpu.SemaphoreType.DMA,
            pltpu.SemaphoreType.DMA,
        ],
    )(src_flat, w_packed, pooled_p)


def kernel(src, W, P):
    b, l = src.shape
    v, d = W.shape
    n_out = b * (l // GRP)
    w_packed = _pack_table(W, 5000)
    pooled_p = _pool_pos(P, (l // GRP) * MAXW, d)
    out = _encode(src.reshape(b * l), w_packed, pooled_p, n_out, d, 16)
    return out.reshape(b, l // GRP, d)


# speedup vs baseline: 3.1541x; 1.5824x over previous
import functools

import jax
import jax.numpy as jnp
from jax import lax
from jax.experimental import pallas as pl
from jax.experimental.pallas import tpu as pltpu
from jax.experimental.pallas import tpu_sc as plsc

AVG = 15
MAXW = 3
GRP = AVG * MAXW
LANES = 16
CHUNK = 120
N_CORES = 2
N_SUB = 16


def _pool_body(p_ref, out_ref):
    x = p_ref[...]
    n, d = out_ref.shape
    out_ref[...] = x.reshape(n, AVG, d).sum(axis=1) * (1.0 / AVG)


def _pool_pos(p, n_sub, d):
    return pl.pallas_call(
        _pool_body,
        in_specs=[pl.BlockSpec((n_sub * AVG, d), lambda: (0, 0))],
        out_specs=pl.BlockSpec((n_sub, d), lambda: (0, 0)),
        out_shape=jax.ShapeDtypeStruct((n_sub, d), jnp.float32),
    )(p[:n_sub * AVG])


def _encoder_body(nq, n_chunks, out_per_w, win_per_w, scale_w,
                  src_ref, w_ref, pp_ref, out_ref,
                  idx, rows, pp, out_v, sem_a, sem_b):
    nc = lax.axis_index("c")
    ns = lax.axis_index("s")
    w_id = nc * N_SUB + ns
    toks_per_w = out_per_w * GRP
    tok0 = w_id * toks_per_w
    half = n_chunks // 2

    copies = []
    for k in range(n_chunks):
        pltpu.sync_copy(src_ref.at[pl.ds(tok0 + k * CHUNK, CHUNK)],
                        idx.at[k])
        sem = sem_a if k < half else sem_b
        copies.append(pltpu.async_copy(
            w_ref.at[idx.at[k]], rows.at[pl.ds(k * CHUNK, CHUNK)], sem))

    pltpu.sync_copy(pp_ref.at[pl.ds((w_id % N_CORES) * win_per_w, win_per_w)],
                    pp)

    def compute(j, _):
        r0 = j * GRP
        res = None
        for kk in range(MAXW):
            b0 = r0 + kk * AVG
            accs = [None] * nq
            for i in range(AVG):
                for q in range(nq):
                    v = rows[b0 + i, pl.ds(q * LANES, LANES)]
                    accs[q] = v if accs[q] is None else accs[q] + v
            es = [accs[q] * scale_w + pp[j * MAXW + kk, pl.ds(q * LANES, LANES)]
                  for q in range(nq)]
            if res is None:
                res = es
            else:
                res = [jnp.maximum(x, y) for x, y in zip(res, es)]
        for q in range(nq):
            out_v[j, pl.ds(q * LANES, LANES)] = res[q]
        return 0

    for k in range(half):
        copies[k].wait()
    j_mid = (half * CHUNK) // GRP
    lax.fori_loop(0, j_mid, compute, 0)

    for k in range(half, n_chunks):
        copies[k].wait()
    lax.fori_loop(j_mid, out_per_w, compute, 0)

    pltpu.sync_copy(out_v, out_ref.at[pl.ds(w_id * out_per_w, out_per_w)])


@functools.partial(jax.jit, static_argnums=(3, 4))
def _encode(src_flat, w, pooled_p, n_out, d):
    n_workers = N_CORES * N_SUB
    out_per_w = n_out // n_workers
    win_per_w = out_per_w * MAXW
    n_chunks = (out_per_w * GRP) // CHUNK
    nq = d // LANES
    scale_w = float(d) ** 0.5 / AVG
    mesh = plsc.VectorSubcoreMesh(core_axis_name="c", subcore_axis_name="s",
                                  num_cores=N_CORES)
    body = functools.partial(_encoder_body, nq, n_chunks, out_per_w,
                             win_per_w, scale_w)
    return pl.kernel(
        body,
        out_type=jax.ShapeDtypeStruct((n_out, d), jnp.float32),
        mesh=mesh,
        compiler_params=pltpu.CompilerParams(use_tc_tiling_on_sc=False,
                                             needs_layout_passes=False),
        scratch_types=[
            pltpu.VMEM((n_chunks, CHUNK), jnp.int32),
            pltpu.VMEM((n_chunks * CHUNK, d), jnp.float32),
            pltpu.VMEM((win_per_w, d), jnp.float32),
            pltpu.VMEM((out_per_w, d), jnp.float32),
            pltpu.SemaphoreType.DMA,
            pltpu.SemaphoreType.DMA,
        ],
    )(src_flat, w, pooled_p)


def kernel(src, W, P):
    b, l = src.shape
    v, d = W.shape
    n_out = b * (l // GRP)
    pooled_p = _pool_pos(P, (l // GRP) * MAXW, d)
    out = _encode(src.reshape(b * l), W, pooled_p, n_out, d)
    return out.reshape(b, l // GRP, d)
